# baseline (speedup 1.0000x reference)
import jax
import jax.numpy as jnp
from jax.experimental import pallas as pl
from jax.experimental.pallas import tpu as pltpu

HASH_DIM = 8
N_HASHES = 4
BLK4 = 512


def _fused_kernel(x_ref, w_ref, rot_ref, u_ref, sim_ref, ort_ref,
                  wb4_ref, rcat4_ref, hcat4_ref):
    half = N_HASHES * HASH_DIM

    @pl.when(pl.program_id(0) == 0)
    def _build():
        w = w_ref[...]
        uvec = u_ref[0, :]
        uh = uvec / (jnp.sqrt(jnp.sum(uvec * uvec)) + 1e-6)
        hm = (jnp.eye(HASH_DIM, dtype=jnp.float32)
              - 2.0 * uh[:, None] * uh[None, :])
        rcat = jnp.concatenate([rot_ref[h] for h in range(N_HASHES)], axis=1)
        hcat = jnp.concatenate([rot_ref[h] @ hm for h in range(N_HASHES)],
                               axis=1)
        wt = w.T
        z = jnp.zeros_like(wt)
        for t in range(4):
            row = jnp.concatenate(
                [wt if s == t else z for s in range(4)], axis=1)
            wb4_ref[t * 1024:(t + 1) * 1024, :] = row
        zr = jnp.zeros_like(rcat)
        for t in range(4):
            rcat4_ref[t * HASH_DIM:(t + 1) * HASH_DIM, :] = jnp.concatenate(
                [rcat if s == t else zr for s in range(4)], axis=1)
            hcat4_ref[t * HASH_DIM:(t + 1) * HASH_DIM, :] = jnp.concatenate(
                [hcat if s == t else zr for s in range(4)], axis=1)

    base4 = jnp.dot(x_ref[...], wb4_ref[...],
                    preferred_element_type=jnp.float32)
    sim_ref[...] = jnp.dot(base4, rcat4_ref[...],
                           preferred_element_type=jnp.float32)
    ort_ref[...] = jnp.dot(base4, hcat4_ref[...],
                           preferred_element_type=jnp.float32)


def kernel(x, W_base, rot, u):
    B, N, D = x.shape
    T = B * N
    T4 = T // 4
    x4 = x.reshape(T4, 4 * D)
    half = N_HASHES * HASH_DIM

    sim2, ort2 = pl.pallas_call(
        _fused_kernel,
        grid=(T4 // BLK4,),
        in_specs=[
            pl.BlockSpec((BLK4, 4 * D), lambda i: (i, 0)),
            pl.BlockSpec((HASH_DIM, D), lambda i: (0, 0)),
            pl.BlockSpec((N_HASHES, HASH_DIM, HASH_DIM), lambda i: (0, 0, 0)),
            pl.BlockSpec((1, HASH_DIM), lambda i: (0, 0)),
        ],
        out_specs=[
            pl.BlockSpec((BLK4, 4 * half), lambda i: (i, 0)),
            pl.BlockSpec((BLK4, 4 * half), lambda i: (i, 0)),
        ],
        out_shape=[
            jax.ShapeDtypeStruct((T4, 4 * half), jnp.float32),
            jax.ShapeDtypeStruct((T4, 4 * half), jnp.float32),
        ],
        scratch_shapes=[
            pltpu.VMEM((4 * D, half), jnp.float32),
            pltpu.VMEM((4 * HASH_DIM, 4 * half), jnp.float32),
            pltpu.VMEM((4 * HASH_DIM, 4 * half), jnp.float32),
        ],
        compiler_params=pltpu.CompilerParams(
            dimension_semantics=("arbitrary",)),
    )(x4, W_base, rot, u.reshape(1, HASH_DIM))

    sim = sim2.reshape(B, N, N_HASHES, HASH_DIM)
    ort = ort2.reshape(B, N, N_HASHES, HASH_DIM)
    return (sim, ort)


# speedup vs baseline: 1.4876x; 1.4876x over previous
import jax
import jax.numpy as jnp
from jax.experimental import pallas as pl
from jax.experimental.pallas import tpu as pltpu

HASH_DIM = 8
N_HASHES = 4
BLK = 4096


def _fused_kernel(x_ref, w_ref, rot_ref, u_ref, sim_ref, ort_ref,
                  ts_ref, to_ref):
    w = w_ref[...]
    uvec = u_ref[0, :]
    uh = uvec / (jnp.sqrt(jnp.sum(uvec * uvec)) + 1e-6)
    hm = jnp.eye(HASH_DIM, dtype=jnp.float32) - 2.0 * uh[:, None] * uh[None, :]
    cats = [rot_ref[h] for h in range(N_HASHES)]
    cats += [rot_ref[h] @ hm for h in range(N_HASHES)]
    ccat = jax.lax.dot_general(
        w, jnp.concatenate(cats, axis=1),
        dimension_numbers=(((0,), (0,)), ((), ())),
        preferred_element_type=jnp.float32)
    half = N_HASHES * HASH_DIM
    out = jnp.dot(x_ref[...], ccat, preferred_element_type=jnp.float32)
    ts_ref[...] = out[:, :half]
    to_ref[...] = out[:, half:]
    sim_ref[...] = jnp.concatenate(
        [ts_ref[t::4, :] for t in range(4)], axis=1)
    ort_ref[...] = jnp.concatenate(
        [to_ref[t::4, :] for t in range(4)], axis=1)


def kernel(x, W_base, rot, u):
    B, N, D = x.shape
    T = B * N
    x2 = x.reshape(T, D)
    half = N_HASHES * HASH_DIM

    sim2, ort2 = pl.pallas_call(
        _fused_kernel,
        grid=(T // BLK,),
        in_specs=[
            pl.BlockSpec((BLK, D), lambda i: (i, 0)),
            pl.BlockSpec((HASH_DIM, D), lambda i: (0, 0)),
            pl.BlockSpec((N_HASHES, HASH_DIM, HASH_DIM), lambda i: (0, 0, 0)),
            pl.BlockSpec((1, HASH_DIM), lambda i: (0, 0)),
        ],
        out_specs=[
            pl.BlockSpec((BLK // 4, 4 * half), lambda i: (i, 0)),
            pl.BlockSpec((BLK // 4, 4 * half), lambda i: (i, 0)),
        ],
        out_shape=[
            jax.ShapeDtypeStruct((T // 4, 4 * half), jnp.float32),
            jax.ShapeDtypeStruct((T // 4, 4 * half), jnp.float32),
        ],
        scratch_shapes=[
            pltpu.VMEM((BLK, half), jnp.float32),
            pltpu.VMEM((BLK, half), jnp.float32),
        ],
        compiler_params=pltpu.CompilerParams(
            dimension_semantics=("arbitrary",)),
    )(x2, W_base, rot, u.reshape(1, HASH_DIM))

    sim = sim2.reshape(B, N, N_HASHES, HASH_DIM)
    ort = ort2.reshape(B, N, N_HASHES, HASH_DIM)
    return (sim, ort)
